# baseline (speedup 1.0000x reference)
import functools

import jax
import jax.numpy as jnp
from jax import lax
from jax.experimental import pallas as pl
from jax.experimental.pallas import tpu as pltpu, tpu_sc as plsc

VOCAB = 1000000
EMB_DIM = 16
BATCH = 16384
FIELDS = 26

_INFO = plsc.get_sparse_core_info()
_NC, _NS = _INFO.num_cores, _INFO.num_subcores
_NW = _NC * _NS
_TOTAL = BATCH * FIELDS
_PER_W = _TOTAL // _NW
_CHUNK = 1664
_NCHUNK = _PER_W // _CHUNK
_NBUF = 2

assert _PER_W * _NW == _TOTAL
assert _NCHUNK * _CHUNK == _PER_W
assert _CHUNK % 8 == 0 and _PER_W % 8 == 0


def _gather_kernel(idx_hbm, *rest):
    cols = rest[:EMB_DIM]
    out_hbm = rest[EMB_DIM]
    idx_v, rows, gsems, osems = rest[EMB_DIM + 1:]
    wid = lax.axis_index("s") * _NC + lax.axis_index("c")
    base = wid * _PER_W
    pltpu.sync_copy(idx_hbm.at[pl.ds(base, _PER_W)], idx_v)

    def start_gathers(c):
        b = c % _NBUF
        idx_c = idx_v.at[pl.ds(c * _CHUNK, _CHUNK)]
        for e in range(EMB_DIM):
            pltpu.async_copy(cols[e].at[idx_c], rows[b].at[e], gsems[b])

    def drain_gathers(c):
        b = c % _NBUF
        idx_c = idx_v.at[pl.ds(c * _CHUNK, _CHUNK)]
        for e in range(EMB_DIM):
            pltpu.make_async_copy(
                cols[e].at[idx_c], rows[b].at[e], gsems[b]).wait()

    def start_out(c):
        b = c % _NBUF
        pltpu.async_copy(
            rows[b], out_hbm.at[:, pl.ds(base + c * _CHUNK, _CHUNK)], osems[b])

    def wait_out(c):
        b = c % _NBUF
        pltpu.make_async_copy(
            rows[b], out_hbm.at[:, pl.ds(base + c * _CHUNK, _CHUNK)],
            osems[b]).wait()

    start_gathers(0)
    for c in range(_NCHUNK):
        drain_gathers(c)
        start_out(c)
        if c + 1 < _NCHUNK:
            if c >= 1:
                wait_out(c - 1)
            start_gathers(c + 1)
    wait_out(_NCHUNK - 2)
    wait_out(_NCHUNK - 1)


@jax.jit
def _embedding_lookup(idx_flat, *cols):
    mesh = plsc.VectorSubcoreMesh(core_axis_name="c", subcore_axis_name="s")
    k = functools.partial(
        pl.kernel,
        mesh=mesh,
        out_type=jax.ShapeDtypeStruct((EMB_DIM, _TOTAL), jnp.float32),
        scratch_types=[
            pltpu.VMEM((_PER_W,), jnp.int32),
            [pltpu.VMEM((EMB_DIM, _CHUNK), jnp.float32) for _ in range(_NBUF)],
            [pltpu.SemaphoreType.DMA for _ in range(_NBUF)],
            [pltpu.SemaphoreType.DMA for _ in range(_NBUF)],
        ],
        compiler_params=pltpu.CompilerParams(use_tc_tiling_on_sc=False),
    )(_gather_kernel)
    return k(idx_flat, *cols)


def kernel(x, table):
    idx_flat = x.reshape(-1).astype(jnp.int32)
    cols = tuple(table[:, e] for e in range(EMB_DIM))
    out_t = _embedding_lookup(idx_flat, *cols)
    return out_t.T.reshape(BATCH, FIELDS, EMB_DIM)


# speedup vs baseline: 1.1346x; 1.1346x over previous
import functools

import jax
import jax.numpy as jnp
from jax import lax
from jax.experimental import pallas as pl
from jax.experimental.pallas import tpu as pltpu, tpu_sc as plsc

VOCAB = 1000000
EMB_DIM = 16
BATCH = 16384
FIELDS = 26

_INFO = plsc.get_sparse_core_info()
_NC, _NS = _INFO.num_cores, _INFO.num_subcores
_NW = _NC * _NS
_TOTAL = BATCH * FIELDS
_PER_W = _TOTAL // _NW
_CHUNK = 3328
_NCHUNK = _PER_W // _CHUNK
_NBUF = 2

assert _PER_W * _NW == _TOTAL
assert _NCHUNK * _CHUNK == _PER_W
assert _CHUNK % 8 == 0 and _PER_W % 8 == 0


def _gather_kernel(idx_hbm, *rest):
    cols = rest[:EMB_DIM]
    out_hbm = rest[EMB_DIM]
    idx_v, rows, gsems, osems = rest[EMB_DIM + 1:]
    wid = lax.axis_index("s") * _NC + lax.axis_index("c")
    base = wid * _PER_W
    pltpu.sync_copy(idx_hbm.at[pl.ds(base, _PER_W)], idx_v)

    def start_gathers(c):
        b = c % _NBUF
        idx_c = idx_v.at[pl.ds(c * _CHUNK, _CHUNK)]
        for e in range(EMB_DIM):
            pltpu.async_copy(cols[e].at[idx_c], rows[b].at[e], gsems[b])

    def drain_gathers(c):
        b = c % _NBUF
        idx_c = idx_v.at[pl.ds(c * _CHUNK, _CHUNK)]
        for e in range(EMB_DIM):
            pltpu.make_async_copy(
                cols[e].at[idx_c], rows[b].at[e], gsems[b]).wait()

    def start_out(c):
        b = c % _NBUF
        pltpu.async_copy(
            rows[b], out_hbm.at[:, pl.ds(base + c * _CHUNK, _CHUNK)], osems[b])

    def wait_out(c):
        b = c % _NBUF
        pltpu.make_async_copy(
            rows[b], out_hbm.at[:, pl.ds(base + c * _CHUNK, _CHUNK)],
            osems[b]).wait()

    start_gathers(0)
    for c in range(_NCHUNK):
        drain_gathers(c)
        start_out(c)
        if c + 1 < _NCHUNK:
            if c >= 1:
                wait_out(c - 1)
            start_gathers(c + 1)
    wait_out(_NCHUNK - 2)
    wait_out(_NCHUNK - 1)


@jax.jit
def _embedding_lookup(idx_flat, *cols):
    mesh = plsc.VectorSubcoreMesh(core_axis_name="c", subcore_axis_name="s")
    k = functools.partial(
        pl.kernel,
        mesh=mesh,
        out_type=jax.ShapeDtypeStruct((EMB_DIM, _TOTAL), jnp.float32),
        scratch_types=[
            pltpu.VMEM((_PER_W,), jnp.int32),
            [pltpu.VMEM((EMB_DIM, _CHUNK), jnp.float32) for _ in range(_NBUF)],
            [pltpu.SemaphoreType.DMA for _ in range(_NBUF)],
            [pltpu.SemaphoreType.DMA for _ in range(_NBUF)],
        ],
        compiler_params=pltpu.CompilerParams(use_tc_tiling_on_sc=False),
    )(_gather_kernel)
    return k(idx_flat, *cols)


def kernel(x, table):
    idx_flat = x.reshape(-1).astype(jnp.int32)
    cols = tuple(table[:, e] for e in range(EMB_DIM))
    out_t = _embedding_lookup(idx_flat, *cols)
    return out_t.reshape(EMB_DIM, BATCH, FIELDS).transpose(1, 2, 0)
